# Initial kernel scaffold; baseline (speedup 1.0000x reference)
import jax
import jax.numpy as jnp
from jax.experimental import pallas as pl


def kernel(features, edge_index, W1, b1, W2, b2):
    raise NotImplementedError("write your pallas kernel here")


# speedup vs baseline: 8.4459x; 8.4459x over previous
import functools

import jax
import jax.numpy as jnp
from jax import lax
from jax.experimental import pallas as pl
from jax.experimental.pallas import tpu as pltpu
from jax.experimental.pallas import tpu_sc as plsc

N_NODES = 10000
N_EDGES = 320000
D_IN = 128
D_HID = 64
D_OUT = 16
ALPHA = 0.1
K_STEPS = 10

NTILES = 16
CHUNK = 128
CHUNKS_PER_TILE = 160
E_PAD = NTILES * CHUNKS_PER_TILE * CHUNK
N_PAD = 10240
DUMP = N_PAD
TBL = N_PAD + 128
ROWS_PER_TILE = N_PAD // NTILES
ZROWS = TBL // NTILES


def _mlp_body(x_ref, w1_ref, b1_ref, w2_ref, b2_ref, o_ref):
    h = jnp.dot(x_ref[...], w1_ref[...], preferred_element_type=jnp.float32)
    h = jnp.maximum(h + b1_ref[...], 0.0)
    o = jnp.dot(h, w2_ref[...], preferred_element_type=jnp.float32)
    o_ref[...] = o + b2_ref[...]


def _mlp(features, W1, b1, W2, b2):
    blk = 2000
    grid = (N_NODES // blk,)
    return pl.pallas_call(
        _mlp_body,
        grid=grid,
        in_specs=[
            pl.BlockSpec((blk, D_IN), lambda i: (i, 0)),
            pl.BlockSpec((D_IN, D_HID), lambda i: (0, 0)),
            pl.BlockSpec((1, D_HID), lambda i: (0, 0)),
            pl.BlockSpec((D_HID, D_OUT), lambda i: (0, 0)),
            pl.BlockSpec((1, D_OUT), lambda i: (0, 0)),
        ],
        out_specs=pl.BlockSpec((blk, D_OUT), lambda i: (i, 0)),
        out_shape=jax.ShapeDtypeStruct((N_NODES, D_OUT), jnp.float32),
    )(features, W1, b1.reshape(1, D_HID), W2, b2.reshape(1, D_OUT))


def _rsqrt16(x):
    i = lax.bitcast_convert_type(x, jnp.int32)
    i = jnp.int32(0x5F3759DF) - lax.shift_right_arithmetic(i, 1)
    y = lax.bitcast_convert_type(i, jnp.float32)
    for _ in range(3):
        y = y * (1.5 - 0.5 * x * y * y)
    return y


def _sc_body(src_hbm, dst_hbm, h0_hbm, out_hbm, featS_hbm,
             srcb, dstb, normO, fI, h0a, zeros, ones, rowsb, aggb, featb,
             fsb, agg_sh):
    tid = lax.axis_index("s")
    ebase = tid * CHUNKS_PER_TILE
    nbase = tid * ROWS_PER_TILE
    zbase = tid * ZROWS

    pltpu.sync_copy(src_hbm.at[pl.ds(ebase, CHUNKS_PER_TILE)], srcb)
    pltpu.sync_copy(dst_hbm.at[pl.ds(ebase, CHUNKS_PER_TILE)], dstb)

    zrow = jnp.zeros((16,), jnp.float32)
    orow = jnp.ones((16,), jnp.float32)

    def _init_z(i, _):
        zeros[i, :] = zrow
        return 0
    lax.fori_loop(0, ZROWS, _init_z, 0)

    def _init_o(i, _):
        ones[i, :] = orow
        return 0
    lax.fori_loop(0, CHUNK, _init_o, 0)

    pltpu.sync_copy(zeros, agg_sh.at[pl.ds(zbase, ZROWS)])
    plsc.subcore_barrier()

    def _degO(j, _):
        pltpu.sync_copy(ones, agg_sh.at[srcb.at[j]], add=True)
        return 0
    lax.fori_loop(0, CHUNKS_PER_TILE, _degO, 0)
    plsc.subcore_barrier()

    pltpu.sync_copy(agg_sh.at[pl.ds(nbase, ROWS_PER_TILE)], aggb)
    pltpu.sync_copy(zeros.at[pl.ds(0, ROWS_PER_TILE)],
                    agg_sh.at[pl.ds(nbase, ROWS_PER_TILE)])

    def _normO(i, _):
        d = jnp.maximum(aggb[i, :], 1.0)
        normO[i, :] = _rsqrt16(d)
        return 0
    lax.fori_loop(0, ROWS_PER_TILE, _normO, 0)

    @pl.when(tid == 0)
    def _zero_dump_deg():
        pltpu.sync_copy(zeros.at[pl.ds(0, TBL - N_PAD)],
                        agg_sh.at[pl.ds(N_PAD, TBL - N_PAD)])
    plsc.subcore_barrier()

    def _degI(j, _):
        pltpu.sync_copy(ones, agg_sh.at[dstb.at[j]], add=True)
        return 0
    lax.fori_loop(0, CHUNKS_PER_TILE, _degI, 0)
    plsc.subcore_barrier()

    pltpu.sync_copy(agg_sh.at[pl.ds(nbase, ROWS_PER_TILE)], aggb)
    pltpu.sync_copy(zeros.at[pl.ds(0, ROWS_PER_TILE)],
                    agg_sh.at[pl.ds(nbase, ROWS_PER_TILE)])

    def _normI(i, _):
        d = jnp.maximum(aggb[i, :], 1.0)
        fI[i, :] = (1.0 - ALPHA) * _rsqrt16(d)
        return 0
    lax.fori_loop(0, ROWS_PER_TILE, _normI, 0)

    @pl.when(tid == 0)
    def _zero_dump_deg2():
        pltpu.sync_copy(zeros.at[pl.ds(0, TBL - N_PAD)],
                        agg_sh.at[pl.ds(N_PAD, TBL - N_PAD)])

    pltpu.sync_copy(h0_hbm.at[pl.ds(nbase, ROWS_PER_TILE)], aggb)

    def _h0(i, _):
        h = aggb[i, :]
        h0a[i, :] = ALPHA * h
        fsb[i, :] = h * normO[i, :]
        return 0
    lax.fori_loop(0, ROWS_PER_TILE, _h0, 0)

    pltpu.sync_copy(fsb, featS_hbm.at[pl.ds(nbase, ROWS_PER_TILE)])

    @pl.when(tid == 0)
    def _zero_dump():
        pltpu.sync_copy(zeros.at[pl.ds(0, TBL - N_PAD)],
                        featS_hbm.at[pl.ds(N_PAD, TBL - N_PAD)])
    plsc.subcore_barrier()

    for s in range(K_STEPS):
        def _edges(j, _):
            pltpu.sync_copy(featS_hbm.at[srcb.at[j]], rowsb)
            pltpu.sync_copy(rowsb, agg_sh.at[dstb.at[j]], add=True)
            return 0
        lax.fori_loop(0, CHUNKS_PER_TILE, _edges, 0)
        plsc.subcore_barrier()

        pltpu.sync_copy(agg_sh.at[pl.ds(nbase, ROWS_PER_TILE)], aggb)
        pltpu.sync_copy(zeros.at[pl.ds(0, ROWS_PER_TILE)],
                        agg_sh.at[pl.ds(nbase, ROWS_PER_TILE)])

        if s < K_STEPS - 1:
            def _node(i, _):
                f = fI[i, :] * aggb[i, :] + h0a[i, :]
                fsb[i, :] = f * normO[i, :]
                return 0
            lax.fori_loop(0, ROWS_PER_TILE, _node, 0)
            pltpu.sync_copy(fsb, featS_hbm.at[pl.ds(nbase, ROWS_PER_TILE)])
        else:
            def _node_last(i, _):
                featb[i, :] = fI[i, :] * aggb[i, :] + h0a[i, :]
                return 0
            lax.fori_loop(0, ROWS_PER_TILE, _node_last, 0)
            pltpu.sync_copy(featb, out_hbm.at[pl.ds(nbase, ROWS_PER_TILE)])
        plsc.subcore_barrier()


@jax.jit
def _appnp(src_p, dst_p, h0):
    mesh = plsc.VectorSubcoreMesh(core_axis_name="c", subcore_axis_name="s",
                                  num_cores=1)
    out, _ = pl.kernel(
        _sc_body,
        out_type=(
            jax.ShapeDtypeStruct((N_PAD, D_OUT), jnp.float32),
            jax.ShapeDtypeStruct((TBL, D_OUT), jnp.float32),
        ),
        mesh=mesh,
        compiler_params=pltpu.CompilerParams(use_tc_tiling_on_sc=False),
        scratch_types=[
            pltpu.VMEM((CHUNKS_PER_TILE, CHUNK), jnp.int32),
            pltpu.VMEM((CHUNKS_PER_TILE, CHUNK), jnp.int32),
            pltpu.VMEM((ROWS_PER_TILE, 16), jnp.float32),
            pltpu.VMEM((ROWS_PER_TILE, 16), jnp.float32),
            pltpu.VMEM((ROWS_PER_TILE, 16), jnp.float32),
            pltpu.VMEM((ZROWS, 16), jnp.float32),
            pltpu.VMEM((CHUNK, 16), jnp.float32),
            pltpu.VMEM((CHUNK, 16), jnp.float32),
            pltpu.VMEM((ROWS_PER_TILE, 16), jnp.float32),
            pltpu.VMEM((ROWS_PER_TILE, 16), jnp.float32),
            pltpu.VMEM((ROWS_PER_TILE, 16), jnp.float32),
            pltpu.VMEM_SHARED((TBL, 16), jnp.float32),
        ],
    )(src_p, dst_p, h0)
    return out


def kernel(features, edge_index, W1, b1, W2, b2):
    h0 = _mlp(features, W1, b1, W2, b2)
    h0p = jnp.concatenate(
        [h0, jnp.zeros((N_PAD - N_NODES, D_OUT), jnp.float32)])
    pad = jnp.full((E_PAD - N_EDGES,), DUMP, dtype=jnp.int32)
    src_p = jnp.concatenate([edge_index[0], pad]).reshape(-1, CHUNK)
    dst_p = jnp.concatenate([edge_index[1], pad]).reshape(-1, CHUNK)
    return _appnp(src_p, dst_p, h0p)[:N_NODES]


# speedup vs baseline: 17.5942x; 2.0832x over previous
import functools

import jax
import jax.numpy as jnp
from jax import lax
from jax.experimental import pallas as pl
from jax.experimental.pallas import tpu as pltpu
from jax.experimental.pallas import tpu_sc as plsc

N_NODES = 10000
N_EDGES = 320000
D_IN = 128
D_HID = 64
D_OUT = 16
ALPHA = 0.1
K_STEPS = 10

NTILES = 16
CHUNK = 128
CHUNKS_PER_TILE = 160
E_PAD = NTILES * CHUNKS_PER_TILE * CHUNK
N_PAD = 10240
DUMP = N_PAD
TBL = N_PAD + 128
ROWS_PER_TILE = N_PAD // NTILES
ZROWS = TBL // NTILES
NBUF = 4
LOOK = 2


def _mlp_body(x_ref, w1_ref, b1_ref, w2_ref, b2_ref, o_ref):
    h = jnp.dot(x_ref[...], w1_ref[...], preferred_element_type=jnp.float32)
    h = jnp.maximum(h + b1_ref[...], 0.0)
    o = jnp.dot(h, w2_ref[...], preferred_element_type=jnp.float32)
    o_ref[...] = o + b2_ref[...]


def _mlp(features, W1, b1, W2, b2):
    blk = 2000
    grid = (N_NODES // blk,)
    return pl.pallas_call(
        _mlp_body,
        grid=grid,
        in_specs=[
            pl.BlockSpec((blk, D_IN), lambda i: (i, 0)),
            pl.BlockSpec((D_IN, D_HID), lambda i: (0, 0)),
            pl.BlockSpec((1, D_HID), lambda i: (0, 0)),
            pl.BlockSpec((D_HID, D_OUT), lambda i: (0, 0)),
            pl.BlockSpec((1, D_OUT), lambda i: (0, 0)),
        ],
        out_specs=pl.BlockSpec((blk, D_OUT), lambda i: (i, 0)),
        out_shape=jax.ShapeDtypeStruct((N_NODES, D_OUT), jnp.float32),
    )(features, W1, b1.reshape(1, D_HID), W2, b2.reshape(1, D_OUT))


def _rsqrt16(x):
    i = lax.bitcast_convert_type(x, jnp.int32)
    i = jnp.int32(0x5F3759DF) - lax.shift_right_arithmetic(i, 1)
    y = lax.bitcast_convert_type(i, jnp.float32)
    for _ in range(3):
        y = y * (1.5 - 0.5 * x * y * y)
    return y


def _sc_body(src_hbm, dst_hbm, h0_hbm, out_hbm, featS_hbm,
             srcb, dstb, normO, fI, h0a, zeros, ones, rowsb, aggb,
             fsb, agg_sh, gsem, ssem, dsem):
    tid = lax.axis_index("s")
    ebase = tid * CHUNKS_PER_TILE
    nbase = tid * ROWS_PER_TILE
    zbase = tid * ZROWS

    pltpu.sync_copy(src_hbm.at[pl.ds(ebase, CHUNKS_PER_TILE)], srcb)
    pltpu.sync_copy(dst_hbm.at[pl.ds(ebase, CHUNKS_PER_TILE)], dstb)

    zrow = jnp.zeros((16,), jnp.float32)
    orow = jnp.ones((16,), jnp.float32)

    def _init_z(i, _):
        zeros[i, :] = zrow
        return 0
    lax.fori_loop(0, ZROWS, _init_z, 0)

    def _init_o(i, _):
        ones[i, :] = orow
        return 0
    lax.fori_loop(0, CHUNK, _init_o, 0)

    pltpu.sync_copy(zeros, agg_sh.at[pl.ds(zbase, ZROWS)])
    plsc.subcore_barrier()

    def _deg_drain(j, _):
        pltpu.make_async_copy(ones, agg_sh.at[pl.ds(0, CHUNK)], dsem).wait()
        return 0

    def _degO(j, _):
        pltpu.async_copy(ones, agg_sh.at[srcb.at[j]], dsem, add=True)

        @pl.when(j >= 8)
        def _lag():
            _deg_drain(j, 0)
        return 0
    lax.fori_loop(0, CHUNKS_PER_TILE, _degO, 0)
    lax.fori_loop(0, 8, _deg_drain, 0)
    plsc.subcore_barrier()

    pltpu.sync_copy(agg_sh.at[pl.ds(nbase, ROWS_PER_TILE)], aggb)
    pltpu.sync_copy(zeros.at[pl.ds(0, ROWS_PER_TILE)],
                    agg_sh.at[pl.ds(nbase, ROWS_PER_TILE)])

    def _normO(i, _):
        d = jnp.maximum(aggb[i, :], 1.0)
        normO[i, :] = _rsqrt16(d)
        return 0
    lax.fori_loop(0, ROWS_PER_TILE, _normO, 0)

    @pl.when(tid == 0)
    def _zero_dump_deg():
        pltpu.sync_copy(zeros.at[pl.ds(0, TBL - N_PAD)],
                        agg_sh.at[pl.ds(N_PAD, TBL - N_PAD)])
    plsc.subcore_barrier()

    def _degI(j, _):
        pltpu.async_copy(ones, agg_sh.at[dstb.at[j]], dsem, add=True)

        @pl.when(j >= 8)
        def _lag():
            _deg_drain(j, 0)
        return 0
    lax.fori_loop(0, CHUNKS_PER_TILE, _degI, 0)
    lax.fori_loop(0, 8, _deg_drain, 0)
    plsc.subcore_barrier()

    pltpu.sync_copy(agg_sh.at[pl.ds(nbase, ROWS_PER_TILE)], aggb)
    pltpu.sync_copy(zeros.at[pl.ds(0, ROWS_PER_TILE)],
                    agg_sh.at[pl.ds(nbase, ROWS_PER_TILE)])

    def _normI(i, _):
        d = jnp.maximum(aggb[i, :], 1.0)
        fI[i, :] = (1.0 - ALPHA) * _rsqrt16(d)
        return 0
    lax.fori_loop(0, ROWS_PER_TILE, _normI, 0)

    @pl.when(tid == 0)
    def _zero_dump_deg2():
        pltpu.sync_copy(zeros.at[pl.ds(0, TBL - N_PAD)],
                        agg_sh.at[pl.ds(N_PAD, TBL - N_PAD)])

    pltpu.sync_copy(h0_hbm.at[pl.ds(nbase, ROWS_PER_TILE)], aggb)

    def _h0(i, _):
        h = aggb[i, :]
        h0a[i, :] = ALPHA * h
        fsb[i, :] = h * normO[i, :]
        return 0
    lax.fori_loop(0, ROWS_PER_TILE, _h0, 0)

    pltpu.sync_copy(fsb, featS_hbm.at[pl.ds(nbase, ROWS_PER_TILE)])

    @pl.when(tid == 0)
    def _zero_dump():
        pltpu.sync_copy(zeros.at[pl.ds(0, TBL - N_PAD)],
                        featS_hbm.at[pl.ds(N_PAD, TBL - N_PAD)])
    plsc.subcore_barrier()

    for s in range(K_STEPS):
        for c in range(LOOK):
            pltpu.async_copy(featS_hbm.at[srcb.at[c]], rowsb.at[c],
                             gsem.at[c])

        def _edges(j, _):
            slot = lax.bitwise_and(j, NBUF - 1)
            nslot = lax.bitwise_and(j + LOOK, NBUF - 1)

            @pl.when(j + LOOK < CHUNKS_PER_TILE)
            def _issue():
                @pl.when(j >= NBUF - LOOK)
                def _wait_prev_scatter():
                    pltpu.make_async_copy(rowsb.at[nslot],
                                          agg_sh.at[pl.ds(0, CHUNK)],
                                          ssem.at[nslot]).wait()
                pltpu.async_copy(featS_hbm.at[srcb.at[j + LOOK]],
                                 rowsb.at[nslot], gsem.at[nslot])

            pltpu.make_async_copy(featS_hbm.at[pl.ds(0, CHUNK)],
                                  rowsb.at[slot], gsem.at[slot]).wait()
            pltpu.async_copy(rowsb.at[slot], agg_sh.at[dstb.at[j]],
                             ssem.at[slot], add=True)
            return 0
        lax.fori_loop(0, CHUNKS_PER_TILE, _edges, 0)
        for c in range(NBUF):
            pltpu.make_async_copy(rowsb.at[c], agg_sh.at[pl.ds(0, CHUNK)],
                                  ssem.at[c]).wait()
        plsc.subcore_barrier()

        pltpu.sync_copy(agg_sh.at[pl.ds(nbase, ROWS_PER_TILE)], aggb)
        pltpu.sync_copy(zeros.at[pl.ds(0, ROWS_PER_TILE)],
                        agg_sh.at[pl.ds(nbase, ROWS_PER_TILE)])

        if s < K_STEPS - 1:
            def _node(i, _):
                f = fI[i, :] * aggb[i, :] + h0a[i, :]
                fsb[i, :] = f * normO[i, :]
                return 0
            lax.fori_loop(0, ROWS_PER_TILE, _node, 0)
            pltpu.sync_copy(fsb, featS_hbm.at[pl.ds(nbase, ROWS_PER_TILE)])
        else:
            def _node_last(i, _):
                fsb[i, :] = fI[i, :] * aggb[i, :] + h0a[i, :]
                return 0
            lax.fori_loop(0, ROWS_PER_TILE, _node_last, 0)
            pltpu.sync_copy(fsb, out_hbm.at[pl.ds(nbase, ROWS_PER_TILE)])
        plsc.subcore_barrier()


@jax.jit
def _appnp(src_p, dst_p, h0):
    mesh = plsc.VectorSubcoreMesh(core_axis_name="c", subcore_axis_name="s",
                                  num_cores=1)
    out, _ = pl.kernel(
        _sc_body,
        out_type=(
            jax.ShapeDtypeStruct((N_PAD, D_OUT), jnp.float32),
            jax.ShapeDtypeStruct((TBL, D_OUT), jnp.float32),
        ),
        mesh=mesh,
        compiler_params=pltpu.CompilerParams(use_tc_tiling_on_sc=False),
        scratch_types=[
            pltpu.VMEM((CHUNKS_PER_TILE, CHUNK), jnp.int32),
            pltpu.VMEM((CHUNKS_PER_TILE, CHUNK), jnp.int32),
            pltpu.VMEM((ROWS_PER_TILE, 16), jnp.float32),
            pltpu.VMEM((ROWS_PER_TILE, 16), jnp.float32),
            pltpu.VMEM((ROWS_PER_TILE, 16), jnp.float32),
            pltpu.VMEM((ZROWS, 16), jnp.float32),
            pltpu.VMEM((CHUNK, 16), jnp.float32),
            pltpu.VMEM((NBUF, CHUNK, 16), jnp.float32),
            pltpu.VMEM((ROWS_PER_TILE, 16), jnp.float32),
            pltpu.VMEM((ROWS_PER_TILE, 16), jnp.float32),
            pltpu.VMEM_SHARED((TBL, 16), jnp.float32),
            pltpu.SemaphoreType.DMA((NBUF,)),
            pltpu.SemaphoreType.DMA((NBUF,)),
            pltpu.SemaphoreType.DMA,
        ],
    )(src_p, dst_p, h0)
    return out


def kernel(features, edge_index, W1, b1, W2, b2):
    h0 = _mlp(features, W1, b1, W2, b2)
    h0p = jnp.concatenate(
        [h0, jnp.zeros((N_PAD - N_NODES, D_OUT), jnp.float32)])
    pad = jnp.full((E_PAD - N_EDGES,), DUMP, dtype=jnp.int32)
    src_p = jnp.concatenate([edge_index[0], pad]).reshape(-1, CHUNK)
    dst_p = jnp.concatenate([edge_index[1], pad]).reshape(-1, CHUNK)
    return _appnp(src_p, dst_p, h0p)[:N_NODES]


# speedup vs baseline: 19.0220x; 1.0812x over previous
import functools

import jax
import jax.numpy as jnp
from jax import lax
from jax.experimental import pallas as pl
from jax.experimental.pallas import tpu as pltpu
from jax.experimental.pallas import tpu_sc as plsc

N_NODES = 10000
N_EDGES = 320000
D_IN = 128
D_HID = 64
D_OUT = 16
ALPHA = 0.1
K_STEPS = 10

NTILES = 16
CHUNK = 128
CHUNKS_PER_TILE = 160
E_PAD = NTILES * CHUNKS_PER_TILE * CHUNK
N_PAD = 10240
DUMP = N_PAD
TBL = N_PAD + 128
ROWS_PER_TILE = N_PAD // NTILES
ZROWS = TBL // NTILES
NBUF = 5
LOOK = 3


def _mlp_body(x_ref, w1_ref, b1_ref, w2_ref, b2_ref, o_ref):
    h = jnp.dot(x_ref[...], w1_ref[...], preferred_element_type=jnp.float32)
    h = jnp.maximum(h + b1_ref[...], 0.0)
    o = jnp.dot(h, w2_ref[...], preferred_element_type=jnp.float32)
    o_ref[...] = o + b2_ref[...]


def _mlp(features, W1, b1, W2, b2):
    blk = 2000
    grid = (N_NODES // blk,)
    return pl.pallas_call(
        _mlp_body,
        grid=grid,
        in_specs=[
            pl.BlockSpec((blk, D_IN), lambda i: (i, 0)),
            pl.BlockSpec((D_IN, D_HID), lambda i: (0, 0)),
            pl.BlockSpec((1, D_HID), lambda i: (0, 0)),
            pl.BlockSpec((D_HID, D_OUT), lambda i: (0, 0)),
            pl.BlockSpec((1, D_OUT), lambda i: (0, 0)),
        ],
        out_specs=pl.BlockSpec((blk, D_OUT), lambda i: (i, 0)),
        out_shape=jax.ShapeDtypeStruct((N_NODES, D_OUT), jnp.float32),
    )(features, W1, b1.reshape(1, D_HID), W2, b2.reshape(1, D_OUT))


def _rsqrt16(x):
    i = lax.bitcast_convert_type(x, jnp.int32)
    i = jnp.int32(0x5F3759DF) - lax.shift_right_arithmetic(i, 1)
    y = lax.bitcast_convert_type(i, jnp.float32)
    for _ in range(3):
        y = y * (1.5 - 0.5 * x * y * y)
    return y


def _sc_body(src_hbm, dst_hbm, h0_hbm, out_hbm, featS_hbm,
             srcb, dstb, normO, fI, h0a, zeros, ones, rowsb, aggb,
             fsb, agg_sh, gsem, ssem, dsem):
    tid = lax.axis_index("s")
    ebase = tid * CHUNKS_PER_TILE
    nbase = tid * ROWS_PER_TILE
    zbase = tid * ZROWS

    pltpu.sync_copy(src_hbm.at[pl.ds(ebase, CHUNKS_PER_TILE)], srcb)
    pltpu.sync_copy(dst_hbm.at[pl.ds(ebase, CHUNKS_PER_TILE)], dstb)

    zrow = jnp.zeros((16,), jnp.float32)
    orow = jnp.ones((16,), jnp.float32)

    @plsc.parallel_loop(0, ZROWS, unroll=4)
    def _init_z(i):
        zeros[i, :] = zrow

    @plsc.parallel_loop(0, CHUNK, unroll=4)
    def _init_o(i):
        ones[i, :] = orow

    pltpu.sync_copy(zeros, agg_sh.at[pl.ds(zbase, ZROWS)])
    plsc.subcore_barrier()

    def _deg_drain(j, _):
        pltpu.make_async_copy(ones, agg_sh.at[pl.ds(0, CHUNK)], dsem).wait()
        return 0

    def _degO(j, _):
        pltpu.async_copy(ones, agg_sh.at[srcb.at[j]], dsem, add=True)

        @pl.when(j >= 8)
        def _lag():
            _deg_drain(j, 0)
        return 0
    lax.fori_loop(0, CHUNKS_PER_TILE, _degO, 0)
    lax.fori_loop(0, 8, _deg_drain, 0)
    plsc.subcore_barrier()

    pltpu.sync_copy(agg_sh.at[pl.ds(nbase, ROWS_PER_TILE)], aggb)
    pltpu.sync_copy(zeros.at[pl.ds(0, ROWS_PER_TILE)],
                    agg_sh.at[pl.ds(nbase, ROWS_PER_TILE)])

    @plsc.parallel_loop(0, ROWS_PER_TILE, unroll=4)
    def _normO(i):
        d = jnp.maximum(aggb[i, :], 1.0)
        normO[i, :] = _rsqrt16(d)

    @pl.when(tid == 0)
    def _zero_dump_deg():
        pltpu.sync_copy(zeros.at[pl.ds(0, TBL - N_PAD)],
                        agg_sh.at[pl.ds(N_PAD, TBL - N_PAD)])
    plsc.subcore_barrier()

    def _degI(j, _):
        pltpu.async_copy(ones, agg_sh.at[dstb.at[j]], dsem, add=True)

        @pl.when(j >= 8)
        def _lag():
            _deg_drain(j, 0)
        return 0
    lax.fori_loop(0, CHUNKS_PER_TILE, _degI, 0)
    lax.fori_loop(0, 8, _deg_drain, 0)
    plsc.subcore_barrier()

    pltpu.sync_copy(agg_sh.at[pl.ds(nbase, ROWS_PER_TILE)], aggb)
    pltpu.sync_copy(zeros.at[pl.ds(0, ROWS_PER_TILE)],
                    agg_sh.at[pl.ds(nbase, ROWS_PER_TILE)])

    @plsc.parallel_loop(0, ROWS_PER_TILE, unroll=4)
    def _normI(i):
        d = jnp.maximum(aggb[i, :], 1.0)
        fI[i, :] = (1.0 - ALPHA) * _rsqrt16(d)

    @pl.when(tid == 0)
    def _zero_dump_deg2():
        pltpu.sync_copy(zeros.at[pl.ds(0, TBL - N_PAD)],
                        agg_sh.at[pl.ds(N_PAD, TBL - N_PAD)])

    pltpu.sync_copy(h0_hbm.at[pl.ds(nbase, ROWS_PER_TILE)], aggb)

    @plsc.parallel_loop(0, ROWS_PER_TILE, unroll=4)
    def _h0(i):
        h = aggb[i, :]
        h0a[i, :] = ALPHA * h
        fsb[i, :] = h * normO[i, :]

    pltpu.sync_copy(fsb, featS_hbm.at[pl.ds(nbase, ROWS_PER_TILE)])

    @pl.when(tid == 0)
    def _zero_dump():
        pltpu.sync_copy(zeros.at[pl.ds(0, TBL - N_PAD)],
                        featS_hbm.at[pl.ds(N_PAD, TBL - N_PAD)])
    plsc.subcore_barrier()

    for s in range(K_STEPS):
        for c in range(LOOK):
            pltpu.async_copy(featS_hbm.at[srcb.at[c]], rowsb.at[c],
                             gsem.at[c])

        def _edges(j, _):
            slot = lax.rem(j, NBUF)
            nslot = lax.rem(j + LOOK, NBUF)

            @pl.when(j + LOOK < CHUNKS_PER_TILE)
            def _issue():
                @pl.when(j >= NBUF - LOOK)
                def _wait_prev_scatter():
                    pltpu.make_async_copy(rowsb.at[nslot],
                                          agg_sh.at[pl.ds(0, CHUNK)],
                                          ssem.at[nslot]).wait()
                pltpu.async_copy(featS_hbm.at[srcb.at[j + LOOK]],
                                 rowsb.at[nslot], gsem.at[nslot])

            pltpu.make_async_copy(featS_hbm.at[pl.ds(0, CHUNK)],
                                  rowsb.at[slot], gsem.at[slot]).wait()
            pltpu.async_copy(rowsb.at[slot], agg_sh.at[dstb.at[j]],
                             ssem.at[slot], add=True)
            return 0
        lax.fori_loop(0, CHUNKS_PER_TILE, _edges, 0)
        for c in range(NBUF):
            pltpu.make_async_copy(rowsb.at[c], agg_sh.at[pl.ds(0, CHUNK)],
                                  ssem.at[c]).wait()
        plsc.subcore_barrier()

        pltpu.sync_copy(agg_sh.at[pl.ds(nbase, ROWS_PER_TILE)], aggb)
        pltpu.sync_copy(zeros.at[pl.ds(0, ROWS_PER_TILE)],
                        agg_sh.at[pl.ds(nbase, ROWS_PER_TILE)])

        if s < K_STEPS - 1:
            @plsc.parallel_loop(0, ROWS_PER_TILE, unroll=4)
            def _node(i):
                f = fI[i, :] * aggb[i, :] + h0a[i, :]
                fsb[i, :] = f * normO[i, :]
            pltpu.sync_copy(fsb, featS_hbm.at[pl.ds(nbase, ROWS_PER_TILE)])
        else:
            @plsc.parallel_loop(0, ROWS_PER_TILE, unroll=4)
            def _node_last(i):
                fsb[i, :] = fI[i, :] * aggb[i, :] + h0a[i, :]
            pltpu.sync_copy(fsb, out_hbm.at[pl.ds(nbase, ROWS_PER_TILE)])
        plsc.subcore_barrier()


@jax.jit
def _appnp(src_p, dst_p, h0):
    mesh = plsc.VectorSubcoreMesh(core_axis_name="c", subcore_axis_name="s",
                                  num_cores=1)
    out, _ = pl.kernel(
        _sc_body,
        out_type=(
            jax.ShapeDtypeStruct((N_PAD, D_OUT), jnp.float32),
            jax.ShapeDtypeStruct((TBL, D_OUT), jnp.float32),
        ),
        mesh=mesh,
        compiler_params=pltpu.CompilerParams(use_tc_tiling_on_sc=False),
        scratch_types=[
            pltpu.VMEM((CHUNKS_PER_TILE, CHUNK), jnp.int32),
            pltpu.VMEM((CHUNKS_PER_TILE, CHUNK), jnp.int32),
            pltpu.VMEM((ROWS_PER_TILE, 16), jnp.float32),
            pltpu.VMEM((ROWS_PER_TILE, 16), jnp.float32),
            pltpu.VMEM((ROWS_PER_TILE, 16), jnp.float32),
            pltpu.VMEM((ZROWS, 16), jnp.float32),
            pltpu.VMEM((CHUNK, 16), jnp.float32),
            pltpu.VMEM((NBUF, CHUNK, 16), jnp.float32),
            pltpu.VMEM((ROWS_PER_TILE, 16), jnp.float32),
            pltpu.VMEM((ROWS_PER_TILE, 16), jnp.float32),
            pltpu.VMEM_SHARED((TBL, 16), jnp.float32),
            pltpu.SemaphoreType.DMA((NBUF,)),
            pltpu.SemaphoreType.DMA((NBUF,)),
            pltpu.SemaphoreType.DMA,
        ],
    )(src_p, dst_p, h0)
    return out


def kernel(features, edge_index, W1, b1, W2, b2):
    h0 = _mlp(features, W1, b1, W2, b2)
    h0p = jnp.concatenate(
        [h0, jnp.zeros((N_PAD - N_NODES, D_OUT), jnp.float32)])
    pad = jnp.full((E_PAD - N_EDGES,), DUMP, dtype=jnp.int32)
    src_p = jnp.concatenate([edge_index[0], pad]).reshape(-1, CHUNK)
    dst_p = jnp.concatenate([edge_index[1], pad]).reshape(-1, CHUNK)
    return _appnp(src_p, dst_p, h0p)[:N_NODES]


# speedup vs baseline: 19.2961x; 1.0144x over previous
import jax
import jax.numpy as jnp
from jax import lax
from jax.experimental import pallas as pl
from jax.experimental.pallas import tpu as pltpu
from jax.experimental.pallas import tpu_sc as plsc

N_NODES = 10000
N_EDGES = 320000
D_IN = 128
D_HID = 64
D_OUT = 16
ALPHA = 0.1
K_STEPS = 10

NCORES = 2
NTILES = 16
NW = NCORES * NTILES
CHUNK = 128
CHUNKS_PER_W = 80
E_PAD = NW * CHUNKS_PER_W * CHUNK
N_PAD = 10240
DUMP = N_PAD
TBL = N_PAD + 128
ROWS_PER_W = N_PAD // NW
ROWS_PER_TILE = N_PAD // NTILES
ZROWS = TBL // NTILES
NBUF = 5
LOOK = 3


def _mlp_body(x_ref, w1_ref, b1_ref, w2_ref, b2_ref, o_ref):
    h = jnp.dot(x_ref[...], w1_ref[...], preferred_element_type=jnp.float32)
    h = jnp.maximum(h + b1_ref[...], 0.0)
    o = jnp.dot(h, w2_ref[...], preferred_element_type=jnp.float32)
    o_ref[...] = o + b2_ref[...]


def _mlp(features, W1, b1, W2, b2):
    blk = 2000
    grid = (N_NODES // blk,)
    return pl.pallas_call(
        _mlp_body,
        grid=grid,
        in_specs=[
            pl.BlockSpec((blk, D_IN), lambda i: (i, 0)),
            pl.BlockSpec((D_IN, D_HID), lambda i: (0, 0)),
            pl.BlockSpec((1, D_HID), lambda i: (0, 0)),
            pl.BlockSpec((D_HID, D_OUT), lambda i: (0, 0)),
            pl.BlockSpec((1, D_OUT), lambda i: (0, 0)),
        ],
        out_specs=pl.BlockSpec((blk, D_OUT), lambda i: (i, 0)),
        out_shape=jax.ShapeDtypeStruct((N_NODES, D_OUT), jnp.float32),
    )(features, W1, b1.reshape(1, D_HID), W2, b2.reshape(1, D_OUT))


def _rsqrt16(x):
    i = lax.bitcast_convert_type(x, jnp.int32)
    i = jnp.int32(0x5F3759DF) - lax.shift_right_arithmetic(i, 1)
    y = lax.bitcast_convert_type(i, jnp.float32)
    for _ in range(3):
        y = y * (1.5 - 0.5 * x * y * y)
    return y


def _sc_body(src_hbm, dst_hbm, h0_hbm, out_hbm,
             featS_hbm, aggH_hbm,
             srcb, dstb, normO, fI, h0a, zeros, ones, rowsb, agg0, agg1,
             fsb, agg_sh, gsem, ssem, dsem, xsem):
    cid = lax.axis_index("c")
    tid = lax.axis_index("s")
    wid = cid * NTILES + tid
    ebase = wid * CHUNKS_PER_W
    wrow = wid * ROWS_PER_W
    nbase = tid * ROWS_PER_TILE
    zbase = tid * ZROWS
    ocid = 1 - cid
    hbase = cid * N_PAD + nbase

    def _xbar():
        plsc.subcore_barrier()
        pltpu.semaphore_signal(xsem, 1, core_index=ocid)
        pl.semaphore_wait(xsem, 1)

    pltpu.sync_copy(src_hbm.at[pl.ds(ebase, CHUNKS_PER_W)], srcb)
    pltpu.sync_copy(dst_hbm.at[pl.ds(ebase, CHUNKS_PER_W)], dstb)

    zrow = jnp.zeros((16,), jnp.float32)
    orow = jnp.ones((16,), jnp.float32)

    @plsc.parallel_loop(0, ZROWS, unroll=4)
    def _init_z(i):
        zeros[i, :] = zrow

    @plsc.parallel_loop(0, CHUNK, unroll=4)
    def _init_o(i):
        ones[i, :] = orow

    pltpu.sync_copy(zeros, agg_sh.at[pl.ds(zbase, ZROWS)])

    @pl.when(wid == 0)
    def _zero_dump_featS():
        pltpu.sync_copy(zeros.at[pl.ds(0, TBL - N_PAD)],
                        featS_hbm.at[pl.ds(N_PAD, TBL - N_PAD)])
    _xbar()

    def _deg_drain(j, _):
        pltpu.make_async_copy(ones, agg_sh.at[pl.ds(0, CHUNK)], dsem).wait()
        return 0

    def _deg_pass(idxb):
        def _fire(j, _):
            pltpu.async_copy(ones, agg_sh.at[idxb.at[j]], dsem, add=True)

            @pl.when(j >= 8)
            def _lag():
                _deg_drain(j, 0)
            return 0
        lax.fori_loop(0, CHUNKS_PER_W, _fire, 0)
        lax.fori_loop(0, 8, _deg_drain, 0)
        plsc.subcore_barrier()
        pltpu.sync_copy(agg_sh.at[pl.ds(nbase, ROWS_PER_TILE)],
                        aggH_hbm.at[pl.ds(hbase, ROWS_PER_TILE)])
        pltpu.sync_copy(zeros.at[pl.ds(0, ROWS_PER_TILE)],
                        agg_sh.at[pl.ds(nbase, ROWS_PER_TILE)])
        _xbar()

    _deg_pass(srcb)
    pltpu.sync_copy(aggH_hbm.at[pl.ds(wrow, ROWS_PER_W)], agg0)
    pltpu.sync_copy(aggH_hbm.at[pl.ds(N_PAD + wrow, ROWS_PER_W)], agg1)

    @plsc.parallel_loop(0, ROWS_PER_W, unroll=4)
    def _normO(i):
        d = jnp.maximum(agg0[i, :] + agg1[i, :], 1.0)
        normO[i, :] = _rsqrt16(d)
    _xbar()

    _deg_pass(dstb)
    pltpu.sync_copy(aggH_hbm.at[pl.ds(wrow, ROWS_PER_W)], agg0)
    pltpu.sync_copy(aggH_hbm.at[pl.ds(N_PAD + wrow, ROWS_PER_W)], agg1)

    @plsc.parallel_loop(0, ROWS_PER_W, unroll=4)
    def _normI(i):
        d = jnp.maximum(agg0[i, :] + agg1[i, :], 1.0)
        fI[i, :] = (1.0 - ALPHA) * _rsqrt16(d)

    pltpu.sync_copy(h0_hbm.at[pl.ds(wrow, ROWS_PER_W)], agg0)

    @plsc.parallel_loop(0, ROWS_PER_W, unroll=4)
    def _h0(i):
        h = agg0[i, :]
        h0a[i, :] = ALPHA * h
        fsb[i, :] = h * normO[i, :]

    pltpu.sync_copy(fsb, featS_hbm.at[pl.ds(wrow, ROWS_PER_W)])
    _xbar()

    for s in range(K_STEPS):
        for c in range(LOOK):
            pltpu.async_copy(featS_hbm.at[srcb.at[c]], rowsb.at[c],
                             gsem.at[c])

        def _edges(j, _):
            slot = lax.rem(j, NBUF)
            nslot = lax.rem(j + LOOK, NBUF)

            @pl.when(j + LOOK < CHUNKS_PER_W)
            def _issue():
                @pl.when(j >= NBUF - LOOK)
                def _wait_prev_scatter():
                    pltpu.make_async_copy(rowsb.at[nslot],
                                          agg_sh.at[pl.ds(0, CHUNK)],
                                          ssem.at[nslot]).wait()
                pltpu.async_copy(featS_hbm.at[srcb.at[j + LOOK]],
                                 rowsb.at[nslot], gsem.at[nslot])

            pltpu.make_async_copy(featS_hbm.at[pl.ds(0, CHUNK)],
                                  rowsb.at[slot], gsem.at[slot]).wait()
            pltpu.async_copy(rowsb.at[slot], agg_sh.at[dstb.at[j]],
                             ssem.at[slot], add=True)
            return 0
        lax.fori_loop(0, CHUNKS_PER_W, _edges, 0)
        for c in range(NBUF):
            pltpu.make_async_copy(rowsb.at[c], agg_sh.at[pl.ds(0, CHUNK)],
                                  ssem.at[c]).wait()
        plsc.subcore_barrier()

        pltpu.sync_copy(agg_sh.at[pl.ds(nbase, ROWS_PER_TILE)],
                        aggH_hbm.at[pl.ds(hbase, ROWS_PER_TILE)])
        pltpu.sync_copy(zeros.at[pl.ds(0, ROWS_PER_TILE)],
                        agg_sh.at[pl.ds(nbase, ROWS_PER_TILE)])
        _xbar()

        pltpu.sync_copy(aggH_hbm.at[pl.ds(wrow, ROWS_PER_W)], agg0)
        pltpu.sync_copy(aggH_hbm.at[pl.ds(N_PAD + wrow, ROWS_PER_W)], agg1)

        if s < K_STEPS - 1:
            @plsc.parallel_loop(0, ROWS_PER_W, unroll=4)
            def _node(i):
                f = fI[i, :] * (agg0[i, :] + agg1[i, :]) + h0a[i, :]
                fsb[i, :] = f * normO[i, :]
            pltpu.sync_copy(fsb, featS_hbm.at[pl.ds(wrow, ROWS_PER_W)])
        else:
            @plsc.parallel_loop(0, ROWS_PER_W, unroll=4)
            def _node_last(i):
                fsb[i, :] = fI[i, :] * (agg0[i, :] + agg1[i, :]) + h0a[i, :]
            pltpu.sync_copy(fsb, out_hbm.at[pl.ds(wrow, ROWS_PER_W)])
        _xbar()


@jax.jit
def _appnp(src_p, dst_p, h0):
    mesh = plsc.VectorSubcoreMesh(core_axis_name="c", subcore_axis_name="s",
                                  num_cores=NCORES)
    return pl.kernel(
        _sc_body,
        out_type=jax.ShapeDtypeStruct((N_PAD, D_OUT), jnp.float32),
        mesh=mesh,
        compiler_params=pltpu.CompilerParams(use_tc_tiling_on_sc=False),
        scratch_types=[
            pltpu.HBM((TBL, D_OUT), jnp.float32),
            pltpu.HBM((2 * N_PAD, D_OUT), jnp.float32),
            pltpu.VMEM((CHUNKS_PER_W, CHUNK), jnp.int32),
            pltpu.VMEM((CHUNKS_PER_W, CHUNK), jnp.int32),
            pltpu.VMEM((ROWS_PER_W, 16), jnp.float32),
            pltpu.VMEM((ROWS_PER_W, 16), jnp.float32),
            pltpu.VMEM((ROWS_PER_W, 16), jnp.float32),
            pltpu.VMEM((ZROWS, 16), jnp.float32),
            pltpu.VMEM((CHUNK, 16), jnp.float32),
            pltpu.VMEM((NBUF, CHUNK, 16), jnp.float32),
            pltpu.VMEM((ROWS_PER_W, 16), jnp.float32),
            pltpu.VMEM((ROWS_PER_W, 16), jnp.float32),
            pltpu.VMEM((ROWS_PER_W, 16), jnp.float32),
            pltpu.VMEM_SHARED((TBL, 16), jnp.float32),
            pltpu.SemaphoreType.DMA((NBUF,)),
            pltpu.SemaphoreType.DMA((NBUF,)),
            pltpu.SemaphoreType.DMA,
            pltpu.SemaphoreType.REGULAR,
        ],
    )(src_p, dst_p, h0)


def kernel(features, edge_index, W1, b1, W2, b2):
    h0 = _mlp(features, W1, b1, W2, b2)
    h0p = jnp.concatenate(
        [h0, jnp.zeros((N_PAD - N_NODES, D_OUT), jnp.float32)])
    pad = jnp.full((E_PAD - N_EDGES,), DUMP, dtype=jnp.int32)
    src_p = jnp.concatenate([edge_index[0], pad]).reshape(-1, CHUNK)
    dst_p = jnp.concatenate([edge_index[1], pad]).reshape(-1, CHUNK)
    return _appnp(src_p, dst_p, h0p)[:N_NODES]


# speedup vs baseline: 19.6106x; 1.0163x over previous
import jax
import jax.numpy as jnp
from jax import lax
from jax.experimental import pallas as pl
from jax.experimental.pallas import tpu as pltpu
from jax.experimental.pallas import tpu_sc as plsc

N_NODES = 10000
N_EDGES = 320000
D_IN = 128
D_HID = 64
D_OUT = 16
ALPHA = 0.1
K_STEPS = 10

NCORES = 2
NTILES = 16
NW = NCORES * NTILES
CHUNK = 128
CHUNKS_PER_W = 80
E_PAD = NW * CHUNKS_PER_W * CHUNK
N_PAD = 10240
DUMP = N_PAD
TBL = N_PAD + 128
ROWS_PER_W = N_PAD // NW
ROWS_PER_TILE = N_PAD // NTILES
ZROWS = TBL // NTILES
NBUF = 12
LOOK = 8


def _mlp_body(x_ref, w1_ref, b1_ref, w2_ref, b2_ref, o_ref):
    h = jnp.dot(x_ref[...], w1_ref[...], preferred_element_type=jnp.float32)
    h = jnp.maximum(h + b1_ref[...], 0.0)
    o = jnp.dot(h, w2_ref[...], preferred_element_type=jnp.float32)
    o_ref[...] = o + b2_ref[...]


def _mlp(features, W1, b1, W2, b2):
    blk = 2000
    grid = (N_NODES // blk,)
    return pl.pallas_call(
        _mlp_body,
        grid=grid,
        in_specs=[
            pl.BlockSpec((blk, D_IN), lambda i: (i, 0)),
            pl.BlockSpec((D_IN, D_HID), lambda i: (0, 0)),
            pl.BlockSpec((1, D_HID), lambda i: (0, 0)),
            pl.BlockSpec((D_HID, D_OUT), lambda i: (0, 0)),
            pl.BlockSpec((1, D_OUT), lambda i: (0, 0)),
        ],
        out_specs=pl.BlockSpec((blk, D_OUT), lambda i: (i, 0)),
        out_shape=jax.ShapeDtypeStruct((N_NODES, D_OUT), jnp.float32),
    )(features, W1, b1.reshape(1, D_HID), W2, b2.reshape(1, D_OUT))


def _rsqrt16(x):
    i = lax.bitcast_convert_type(x, jnp.int32)
    i = jnp.int32(0x5F3759DF) - lax.shift_right_arithmetic(i, 1)
    y = lax.bitcast_convert_type(i, jnp.float32)
    for _ in range(3):
        y = y * (1.5 - 0.5 * x * y * y)
    return y


def _sc_body(src_hbm, dst_hbm, h0_hbm, out_hbm,
             featS_hbm, aggH_hbm,
             srcb, dstb, normO, fI, h0a, zeros, ones, rowsb, agg0, agg1,
             fsb, agg_sh, gsem, ssem, dsem, xsem):
    cid = lax.axis_index("c")
    tid = lax.axis_index("s")
    wid = cid * NTILES + tid
    ebase = wid * CHUNKS_PER_W
    wrow = wid * ROWS_PER_W
    nbase = tid * ROWS_PER_TILE
    zbase = tid * ZROWS
    ocid = 1 - cid
    hbase = cid * N_PAD + nbase

    def _xbar():
        plsc.subcore_barrier()
        pltpu.semaphore_signal(xsem, 1, core_index=ocid)
        pl.semaphore_wait(xsem, 1)

    pltpu.sync_copy(src_hbm.at[pl.ds(ebase, CHUNKS_PER_W)], srcb)
    pltpu.sync_copy(dst_hbm.at[pl.ds(ebase, CHUNKS_PER_W)], dstb)

    zrow = jnp.zeros((16,), jnp.float32)
    orow = jnp.ones((16,), jnp.float32)

    @plsc.parallel_loop(0, ZROWS, unroll=4)
    def _init_z(i):
        zeros[i, :] = zrow

    @plsc.parallel_loop(0, CHUNK, unroll=4)
    def _init_o(i):
        ones[i, :] = orow

    pltpu.sync_copy(zeros, agg_sh.at[pl.ds(zbase, ZROWS)])

    @pl.when(wid == 0)
    def _zero_dump_featS():
        pltpu.sync_copy(zeros.at[pl.ds(0, TBL - N_PAD)],
                        featS_hbm.at[pl.ds(N_PAD, TBL - N_PAD)])
    _xbar()

    def _deg_drain(j, _):
        pltpu.make_async_copy(ones, agg_sh.at[pl.ds(0, CHUNK)], dsem).wait()
        return 0

    def _deg_pass(idxb):
        def _fire(j, _):
            pltpu.async_copy(ones, agg_sh.at[idxb.at[j]], dsem, add=True)

            @pl.when(j >= 24)
            def _lag():
                _deg_drain(j, 0)
            return 0
        lax.fori_loop(0, CHUNKS_PER_W, _fire, 0)
        lax.fori_loop(0, 24, _deg_drain, 0)
        plsc.subcore_barrier()
        pltpu.sync_copy(agg_sh.at[pl.ds(nbase, ROWS_PER_TILE)],
                        aggH_hbm.at[pl.ds(hbase, ROWS_PER_TILE)])
        pltpu.sync_copy(zeros.at[pl.ds(0, ROWS_PER_TILE)],
                        agg_sh.at[pl.ds(nbase, ROWS_PER_TILE)])
        _xbar()

    _deg_pass(srcb)
    pltpu.sync_copy(aggH_hbm.at[pl.ds(wrow, ROWS_PER_W)], agg0)
    pltpu.sync_copy(aggH_hbm.at[pl.ds(N_PAD + wrow, ROWS_PER_W)], agg1)

    @plsc.parallel_loop(0, ROWS_PER_W, unroll=4)
    def _normO(i):
        d = jnp.maximum(agg0[i, :] + agg1[i, :], 1.0)
        normO[i, :] = _rsqrt16(d)
    _xbar()

    _deg_pass(dstb)
    pltpu.sync_copy(aggH_hbm.at[pl.ds(wrow, ROWS_PER_W)], agg0)
    pltpu.sync_copy(aggH_hbm.at[pl.ds(N_PAD + wrow, ROWS_PER_W)], agg1)

    @plsc.parallel_loop(0, ROWS_PER_W, unroll=4)
    def _normI(i):
        d = jnp.maximum(agg0[i, :] + agg1[i, :], 1.0)
        fI[i, :] = (1.0 - ALPHA) * _rsqrt16(d)

    pltpu.sync_copy(h0_hbm.at[pl.ds(wrow, ROWS_PER_W)], agg0)

    @plsc.parallel_loop(0, ROWS_PER_W, unroll=4)
    def _h0(i):
        h = agg0[i, :]
        h0a[i, :] = ALPHA * h
        fsb[i, :] = h * normO[i, :]

    pltpu.sync_copy(fsb, featS_hbm.at[pl.ds(wrow, ROWS_PER_W)])
    _xbar()

    for s in range(K_STEPS):
        for c in range(LOOK):
            pltpu.async_copy(featS_hbm.at[srcb.at[c]], rowsb.at[c],
                             gsem.at[c])

        def _edges(j, _):
            slot = lax.rem(j, NBUF)
            nslot = lax.rem(j + LOOK, NBUF)

            @pl.when(j + LOOK < CHUNKS_PER_W)
            def _issue():
                @pl.when(j >= NBUF - LOOK)
                def _wait_prev_scatter():
                    pltpu.make_async_copy(rowsb.at[nslot],
                                          agg_sh.at[pl.ds(0, CHUNK)],
                                          ssem.at[nslot]).wait()
                pltpu.async_copy(featS_hbm.at[srcb.at[j + LOOK]],
                                 rowsb.at[nslot], gsem.at[nslot])

            pltpu.make_async_copy(featS_hbm.at[pl.ds(0, CHUNK)],
                                  rowsb.at[slot], gsem.at[slot]).wait()
            pltpu.async_copy(rowsb.at[slot], agg_sh.at[dstb.at[j]],
                             ssem.at[slot], add=True)
            return 0
        lax.fori_loop(0, CHUNKS_PER_W, _edges, 0)
        for c in range(NBUF):
            pltpu.make_async_copy(rowsb.at[c], agg_sh.at[pl.ds(0, CHUNK)],
                                  ssem.at[c]).wait()
        plsc.subcore_barrier()

        pltpu.sync_copy(agg_sh.at[pl.ds(nbase, ROWS_PER_TILE)],
                        aggH_hbm.at[pl.ds(hbase, ROWS_PER_TILE)])
        pltpu.sync_copy(zeros.at[pl.ds(0, ROWS_PER_TILE)],
                        agg_sh.at[pl.ds(nbase, ROWS_PER_TILE)])
        _xbar()

        pltpu.sync_copy(aggH_hbm.at[pl.ds(wrow, ROWS_PER_W)], agg0)
        pltpu.sync_copy(aggH_hbm.at[pl.ds(N_PAD + wrow, ROWS_PER_W)], agg1)

        if s < K_STEPS - 1:
            @plsc.parallel_loop(0, ROWS_PER_W, unroll=4)
            def _node(i):
                f = fI[i, :] * (agg0[i, :] + agg1[i, :]) + h0a[i, :]
                fsb[i, :] = f * normO[i, :]
            pltpu.sync_copy(fsb, featS_hbm.at[pl.ds(wrow, ROWS_PER_W)])
        else:
            @plsc.parallel_loop(0, ROWS_PER_W, unroll=4)
            def _node_last(i):
                fsb[i, :] = fI[i, :] * (agg0[i, :] + agg1[i, :]) + h0a[i, :]
            pltpu.sync_copy(fsb, out_hbm.at[pl.ds(wrow, ROWS_PER_W)])
        _xbar()


@jax.jit
def _appnp(src_p, dst_p, h0):
    mesh = plsc.VectorSubcoreMesh(core_axis_name="c", subcore_axis_name="s",
                                  num_cores=NCORES)
    return pl.kernel(
        _sc_body,
        out_type=jax.ShapeDtypeStruct((N_PAD, D_OUT), jnp.float32),
        mesh=mesh,
        compiler_params=pltpu.CompilerParams(use_tc_tiling_on_sc=False),
        scratch_types=[
            pltpu.HBM((TBL, D_OUT), jnp.float32),
            pltpu.HBM((2 * N_PAD, D_OUT), jnp.float32),
            pltpu.VMEM((CHUNKS_PER_W, CHUNK), jnp.int32),
            pltpu.VMEM((CHUNKS_PER_W, CHUNK), jnp.int32),
            pltpu.VMEM((ROWS_PER_W, 16), jnp.float32),
            pltpu.VMEM((ROWS_PER_W, 16), jnp.float32),
            pltpu.VMEM((ROWS_PER_W, 16), jnp.float32),
            pltpu.VMEM((ZROWS, 16), jnp.float32),
            pltpu.VMEM((CHUNK, 16), jnp.float32),
            pltpu.VMEM((NBUF, CHUNK, 16), jnp.float32),
            pltpu.VMEM((ROWS_PER_W, 16), jnp.float32),
            pltpu.VMEM((ROWS_PER_W, 16), jnp.float32),
            pltpu.VMEM((ROWS_PER_W, 16), jnp.float32),
            pltpu.VMEM_SHARED((TBL, 16), jnp.float32),
            pltpu.SemaphoreType.DMA((NBUF,)),
            pltpu.SemaphoreType.DMA((NBUF,)),
            pltpu.SemaphoreType.DMA,
            pltpu.SemaphoreType.REGULAR,
        ],
    )(src_p, dst_p, h0)


def kernel(features, edge_index, W1, b1, W2, b2):
    h0 = _mlp(features, W1, b1, W2, b2)
    h0p = jnp.concatenate(
        [h0, jnp.zeros((N_PAD - N_NODES, D_OUT), jnp.float32)])
    pad = jnp.full((E_PAD - N_EDGES,), DUMP, dtype=jnp.int32)
    src_p = jnp.concatenate([edge_index[0], pad]).reshape(-1, CHUNK)
    dst_p = jnp.concatenate([edge_index[1], pad]).reshape(-1, CHUNK)
    return _appnp(src_p, dst_p, h0p)[:N_NODES]


# speedup vs baseline: 42.1346x; 2.1486x over previous
import jax
import jax.numpy as jnp
from jax import lax
from jax.experimental import pallas as pl
from jax.experimental.pallas import tpu as pltpu
from jax.experimental.pallas import tpu_sc as plsc

N_NODES = 10000
N_EDGES = 320000
D_IN = 128
D_HID = 64
D_OUT = 16
ALPHA = 0.1
K_STEPS = 10

NCORES = 2
NTILES = 16
NW = NCORES * NTILES
CHUNK = 128
CHUNKS_PER_W = 80
E_PAD = NW * CHUNKS_PER_W * CHUNK
N_PAD = 10240
DUMP = N_PAD
TBL = N_PAD + 128
ROWS_PER_W = N_PAD // NW
ROWS_PER_TILE = N_PAD // NTILES
ZROWS = TBL // NTILES
NBUF = 12
LOOK = 8


def _mlp_body(x_ref, w1_ref, b1_ref, w2_ref, b2_ref, o_ref):
    h = jnp.dot(x_ref[...], w1_ref[...], preferred_element_type=jnp.float32)
    h = jnp.maximum(h + b1_ref[...], 0.0)
    o = jnp.dot(h, w2_ref[...], preferred_element_type=jnp.float32)
    o_ref[...] = o + b2_ref[...]


def _mlp(features, W1, b1, W2, b2):
    blk = 2000
    grid = (N_NODES // blk,)
    return pl.pallas_call(
        _mlp_body,
        grid=grid,
        in_specs=[
            pl.BlockSpec((blk, D_IN), lambda i: (i, 0)),
            pl.BlockSpec((D_IN, D_HID), lambda i: (0, 0)),
            pl.BlockSpec((1, D_HID), lambda i: (0, 0)),
            pl.BlockSpec((D_HID, D_OUT), lambda i: (0, 0)),
            pl.BlockSpec((1, D_OUT), lambda i: (0, 0)),
        ],
        out_specs=pl.BlockSpec((blk, D_OUT), lambda i: (i, 0)),
        out_shape=jax.ShapeDtypeStruct((N_NODES, D_OUT), jnp.float32),
    )(features, W1, b1.reshape(1, D_HID), W2, b2.reshape(1, D_OUT))


def _rsqrt16(x):
    i = lax.bitcast_convert_type(x, jnp.int32)
    i = jnp.int32(0x5F3759DF) - lax.shift_right_arithmetic(i, 1)
    y = lax.bitcast_convert_type(i, jnp.float32)
    for _ in range(3):
        y = y * (1.5 - 0.5 * x * y * y)
    return y


def _sc_body(src_hbm, dst_hbm, h0_hbm, out_hbm,
             featS_hbm, aggH_hbm,
             srcb, dstb, normO, fI, h0a, zeros, ones, rowsb, agg0, agg1,
             fsb, agg_sh, featS_sh, gsem, ssem, dsem, xsem):
    cid = lax.axis_index("c")
    tid = lax.axis_index("s")
    wid = cid * NTILES + tid
    ebase = wid * CHUNKS_PER_W
    wrow = wid * ROWS_PER_W
    nbase = tid * ROWS_PER_TILE
    zbase = tid * ZROWS
    ocid = 1 - cid
    hbase = cid * N_PAD + nbase

    def _xbar():
        plsc.subcore_barrier()
        pltpu.semaphore_signal(xsem, 1, core_index=ocid)
        pl.semaphore_wait(xsem, 1)

    pltpu.sync_copy(src_hbm.at[pl.ds(ebase, CHUNKS_PER_W)], srcb)
    pltpu.sync_copy(dst_hbm.at[pl.ds(ebase, CHUNKS_PER_W)], dstb)

    zrow = jnp.zeros((16,), jnp.float32)
    orow = jnp.ones((16,), jnp.float32)

    @plsc.parallel_loop(0, ZROWS, unroll=4)
    def _init_z(i):
        zeros[i, :] = zrow

    @plsc.parallel_loop(0, CHUNK, unroll=4)
    def _init_o(i):
        ones[i, :] = orow

    pltpu.sync_copy(zeros, agg_sh.at[pl.ds(zbase, ZROWS)])

    @pl.when(wid == 0)
    def _zero_dump_featS():
        pltpu.sync_copy(zeros.at[pl.ds(0, TBL - N_PAD)],
                        featS_hbm.at[pl.ds(N_PAD, TBL - N_PAD)])
    _xbar()

    def _deg_drain(j, _):
        pltpu.make_async_copy(ones, agg_sh.at[pl.ds(0, CHUNK)], dsem).wait()
        return 0

    def _deg_pass(idxb):
        def _fire(j, _):
            pltpu.async_copy(ones, agg_sh.at[idxb.at[j]], dsem, add=True)

            @pl.when(j >= 24)
            def _lag():
                _deg_drain(j, 0)
            return 0
        lax.fori_loop(0, CHUNKS_PER_W, _fire, 0)
        lax.fori_loop(0, 24, _deg_drain, 0)
        plsc.subcore_barrier()
        pltpu.sync_copy(agg_sh.at[pl.ds(nbase, ROWS_PER_TILE)],
                        aggH_hbm.at[pl.ds(hbase, ROWS_PER_TILE)])
        pltpu.sync_copy(zeros.at[pl.ds(0, ROWS_PER_TILE)],
                        agg_sh.at[pl.ds(nbase, ROWS_PER_TILE)])
        _xbar()

    _deg_pass(srcb)
    pltpu.sync_copy(aggH_hbm.at[pl.ds(wrow, ROWS_PER_W)], agg0)
    pltpu.sync_copy(aggH_hbm.at[pl.ds(N_PAD + wrow, ROWS_PER_W)], agg1)

    @plsc.parallel_loop(0, ROWS_PER_W, unroll=4)
    def _normO(i):
        d = jnp.maximum(agg0[i, :] + agg1[i, :], 1.0)
        normO[i, :] = _rsqrt16(d)
    _xbar()

    _deg_pass(dstb)
    pltpu.sync_copy(aggH_hbm.at[pl.ds(wrow, ROWS_PER_W)], agg0)
    pltpu.sync_copy(aggH_hbm.at[pl.ds(N_PAD + wrow, ROWS_PER_W)], agg1)

    @plsc.parallel_loop(0, ROWS_PER_W, unroll=4)
    def _normI(i):
        d = jnp.maximum(agg0[i, :] + agg1[i, :], 1.0)
        fI[i, :] = (1.0 - ALPHA) * _rsqrt16(d)

    pltpu.sync_copy(h0_hbm.at[pl.ds(wrow, ROWS_PER_W)], agg0)

    @plsc.parallel_loop(0, ROWS_PER_W, unroll=4)
    def _h0(i):
        h = agg0[i, :]
        h0a[i, :] = ALPHA * h
        fsb[i, :] = h * normO[i, :]

    pltpu.sync_copy(fsb, featS_hbm.at[pl.ds(wrow, ROWS_PER_W)])
    _xbar()

    def _feat_in():
        pltpu.sync_copy(featS_hbm.at[pl.ds(zbase, ZROWS)],
                        featS_sh.at[pl.ds(zbase, ZROWS)])
        plsc.subcore_barrier()

    _feat_in()

    for s in range(K_STEPS):
        for c in range(LOOK):
            pltpu.async_copy(featS_sh.at[srcb.at[c]], rowsb.at[c],
                             gsem.at[c])

        def _edges(j, _):
            slot = lax.rem(j, NBUF)
            nslot = lax.rem(j + LOOK, NBUF)

            @pl.when(j + LOOK < CHUNKS_PER_W)
            def _issue():
                @pl.when(j >= NBUF - LOOK)
                def _wait_prev_scatter():
                    pltpu.make_async_copy(rowsb.at[nslot],
                                          agg_sh.at[pl.ds(0, CHUNK)],
                                          ssem.at[nslot]).wait()
                pltpu.async_copy(featS_sh.at[srcb.at[j + LOOK]],
                                 rowsb.at[nslot], gsem.at[nslot])

            pltpu.make_async_copy(featS_hbm.at[pl.ds(0, CHUNK)],
                                  rowsb.at[slot], gsem.at[slot]).wait()
            pltpu.async_copy(rowsb.at[slot], agg_sh.at[dstb.at[j]],
                             ssem.at[slot], add=True)
            return 0
        lax.fori_loop(0, CHUNKS_PER_W, _edges, 0)
        for c in range(NBUF):
            pltpu.make_async_copy(rowsb.at[c], agg_sh.at[pl.ds(0, CHUNK)],
                                  ssem.at[c]).wait()
        plsc.subcore_barrier()

        pltpu.sync_copy(agg_sh.at[pl.ds(nbase, ROWS_PER_TILE)],
                        aggH_hbm.at[pl.ds(hbase, ROWS_PER_TILE)])
        pltpu.sync_copy(zeros.at[pl.ds(0, ROWS_PER_TILE)],
                        agg_sh.at[pl.ds(nbase, ROWS_PER_TILE)])
        _xbar()

        pltpu.sync_copy(aggH_hbm.at[pl.ds(wrow, ROWS_PER_W)], agg0)
        pltpu.sync_copy(aggH_hbm.at[pl.ds(N_PAD + wrow, ROWS_PER_W)], agg1)

        if s < K_STEPS - 1:
            @plsc.parallel_loop(0, ROWS_PER_W, unroll=4)
            def _node(i):
                f = fI[i, :] * (agg0[i, :] + agg1[i, :]) + h0a[i, :]
                fsb[i, :] = f * normO[i, :]
            pltpu.sync_copy(fsb, featS_hbm.at[pl.ds(wrow, ROWS_PER_W)])
            _xbar()
            _feat_in()
        else:
            @plsc.parallel_loop(0, ROWS_PER_W, unroll=4)
            def _node_last(i):
                fsb[i, :] = fI[i, :] * (agg0[i, :] + agg1[i, :]) + h0a[i, :]
            pltpu.sync_copy(fsb, out_hbm.at[pl.ds(wrow, ROWS_PER_W)])
            _xbar()


@jax.jit
def _appnp(src_p, dst_p, h0):
    mesh = plsc.VectorSubcoreMesh(core_axis_name="c", subcore_axis_name="s",
                                  num_cores=NCORES)
    return pl.kernel(
        _sc_body,
        out_type=jax.ShapeDtypeStruct((N_PAD, D_OUT), jnp.float32),
        mesh=mesh,
        compiler_params=pltpu.CompilerParams(use_tc_tiling_on_sc=False),
        scratch_types=[
            pltpu.HBM((TBL, D_OUT), jnp.float32),
            pltpu.HBM((2 * N_PAD, D_OUT), jnp.float32),
            pltpu.VMEM((CHUNKS_PER_W, CHUNK), jnp.int32),
            pltpu.VMEM((CHUNKS_PER_W, CHUNK), jnp.int32),
            pltpu.VMEM((ROWS_PER_W, 16), jnp.float32),
            pltpu.VMEM((ROWS_PER_W, 16), jnp.float32),
            pltpu.VMEM((ROWS_PER_W, 16), jnp.float32),
            pltpu.VMEM((ZROWS, 16), jnp.float32),
            pltpu.VMEM((CHUNK, 16), jnp.float32),
            pltpu.VMEM((NBUF, CHUNK, 16), jnp.float32),
            pltpu.VMEM((ROWS_PER_W, 16), jnp.float32),
            pltpu.VMEM((ROWS_PER_W, 16), jnp.float32),
            pltpu.VMEM((ROWS_PER_W, 16), jnp.float32),
            pltpu.VMEM_SHARED((TBL, 16), jnp.float32),
            pltpu.VMEM_SHARED((TBL, 16), jnp.float32),
            pltpu.SemaphoreType.DMA((NBUF,)),
            pltpu.SemaphoreType.DMA((NBUF,)),
            pltpu.SemaphoreType.DMA,
            pltpu.SemaphoreType.REGULAR,
        ],
    )(src_p, dst_p, h0)


def kernel(features, edge_index, W1, b1, W2, b2):
    h0 = _mlp(features, W1, b1, W2, b2)
    h0p = jnp.concatenate(
        [h0, jnp.zeros((N_PAD - N_NODES, D_OUT), jnp.float32)])
    pad = jnp.full((E_PAD - N_EDGES,), DUMP, dtype=jnp.int32)
    src_p = jnp.concatenate([edge_index[0], pad]).reshape(-1, CHUNK)
    dst_p = jnp.concatenate([edge_index[1], pad]).reshape(-1, CHUNK)
    return _appnp(src_p, dst_p, h0p)[:N_NODES]
